# baseline (speedup 1.0000x reference)
import functools

import jax
import jax.numpy as jnp
from jax import lax
from jax.experimental import pallas as pl
from jax.experimental.pallas import tpu as pltpu
from jax.experimental.pallas import tpu_sc as plsc

_N = 32768
_K = 128
_C = 8
_BN = 16384
_NB = _N // _BN
_EPS = 1e-8
_NS = 16
_CA = _N // _NS
_CB = _N // 32


def _sc_d2_body(xs_hbm, ys_hbm, zs_hbm, lbl_hbm, d2_hbm,
                xs_v, ys_v, zs_v, lbl_v, d2_v, tbl_v, mtab_v, pw_v, red_v,
                shared):
    s = lax.axis_index("s")
    c = lax.axis_index("c")
    base_a = s * _CA
    pltpu.sync_copy(lbl_hbm.at[pl.ds(base_a, _CA)], lbl_v)
    pltpu.sync_copy(xs_hbm.at[pl.ds(base_a, _CA)], xs_v)
    pltpu.sync_copy(ys_hbm.at[pl.ds(base_a, _CA)], ys_v)
    pltpu.sync_copy(zs_hbm.at[pl.ds(base_a, _CA)], zs_v)
    iota = lax.iota(jnp.int32, 16)
    zeros = jnp.zeros((16,), jnp.float32)
    ones = jnp.ones((16,), jnp.float32)

    def zbody(i, _):
        tbl_v[pl.ds(i * 16, 16)] = zeros
        return 0

    lax.fori_loop(0, 32, zbody, 0)

    def abody(i, _):
        sl = pl.ds(i * 16, 16)
        lbl = lbl_v[sl]
        x = xs_v[sl]
        y = ys_v[sl]
        z = zs_v[sl]
        idx = lbl * 16 + iota
        plsc.addupdate_scatter(tbl_v, [idx], ones)
        plsc.addupdate_scatter(tbl_v, [idx + 128], x)
        plsc.addupdate_scatter(tbl_v, [idx + 256], y)
        plsc.addupdate_scatter(tbl_v, [idx + 384], z)
        return 0

    lax.fori_loop(0, _CA // 16, abody, 0)

    base16 = (iota & 7) * 16 + jnp.where(iota >= 8, 128, 0)
    pw0 = zeros
    pw1 = zeros
    for jj in range(16):
        pw0 = pw0 + plsc.load_gather(tbl_v, [base16 + jj])
        pw1 = pw1 + plsc.load_gather(tbl_v, [base16 + (jj + 256)])
    pw_v[pl.ds(0, 16)] = pw0
    pw_v[pl.ds(16, 16)] = pw1
    pltpu.sync_copy(pw_v, shared.at[pl.ds(s * 32, 32)])
    plsc.subcore_barrier()
    pltpu.sync_copy(shared, red_v)

    def rbody(w, carry):
        a0, a1 = carry
        a0 = a0 + red_v[pl.ds(w * 32, 16)]
        a1 = a1 + red_v[pl.ds(w * 32 + 16, 16)]
        return (a0, a1)

    acc0, acc1 = lax.fori_loop(0, _NS, rbody, (zeros, zeros))
    mtab_v[pl.ds(0, 16)] = acc0
    mtab_v[pl.ds(16, 16)] = acc1
    lo = iota & 7
    cnt_rep = plsc.load_gather(mtab_v, [lo])
    sx_rep = plsc.load_gather(mtab_v, [lo + 8])
    sy_rep = plsc.load_gather(mtab_v, [lo + 16])
    sz_rep = plsc.load_gather(mtab_v, [lo + 24])
    pos = cnt_rep > 0.0
    safe = jnp.where(pos, cnt_rep, ones)
    mtab_v[pl.ds(32, 16)] = jnp.where(pos, sx_rep / safe, zeros)
    mtab_v[pl.ds(48, 16)] = jnp.where(pos, sy_rep / safe, zeros)
    mtab_v[pl.ds(64, 16)] = jnp.where(pos, sz_rep / safe, zeros)

    off = c * _CB

    def bbody(i, _):
        sl = pl.ds(off + i * 16, 16)
        lbl = lbl_v[sl]
        x = xs_v[sl]
        y = ys_v[sl]
        z = zs_v[sl]
        mx = plsc.load_gather(mtab_v, [lbl + 32])
        my = plsc.load_gather(mtab_v, [lbl + 48])
        mz = plsc.load_gather(mtab_v, [lbl + 64])
        dx = x - mx
        dy = y - my
        dz = z - mz
        d2_v[pl.ds(i * 16, 16)] = dx * dx + dy * dy + dz * dz
        return 0

    lax.fori_loop(0, _CB // 16, bbody, 0)
    pltpu.sync_copy(d2_v, d2_hbm.at[pl.ds(base_a + off, _CB)])


@functools.lru_cache(maxsize=1)
def _get_sc_d2():
    return pl.kernel(
        _sc_d2_body,
        out_type=jax.ShapeDtypeStruct((_N,), jnp.float32),
        mesh=plsc.VectorSubcoreMesh(core_axis_name="c", subcore_axis_name="s"),
        scratch_types=[
            pltpu.VMEM((_CA,), jnp.float32),
            pltpu.VMEM((_CA,), jnp.float32),
            pltpu.VMEM((_CA,), jnp.float32),
            pltpu.VMEM((_CA,), jnp.int32),
            pltpu.VMEM((_CB,), jnp.float32),
            pltpu.VMEM((512,), jnp.float32),
            pltpu.VMEM((96,), jnp.float32),
            pltpu.VMEM((32,), jnp.float32),
            pltpu.VMEM((512,), jnp.float32),
            pltpu.VMEM_SHARED((512,), jnp.float32),
        ],
        compiler_params=pltpu.CompilerParams(needs_layout_passes=False),
    )


def _tc_body(e_ref, lbl_ref, d2_ref, out_ref, sw_scr, cg_scr):
    j = pl.program_id(0)

    @pl.when(j == 0)
    def _init():
        sw_scr[...] = jnp.zeros_like(sw_scr)
        cg_scr[...] = jnp.zeros_like(cg_scr)

    jstart = pl.multiple_of(j * _BN, _BN)
    eb = e_ref[:, pl.ds(jstart, _BN)]
    lbl = lbl_ref[:, pl.ds(jstart, _BN)]
    d2 = d2_ref[:, pl.ds(jstart, _BN)]
    cid = lax.broadcasted_iota(jnp.int32, (_C, _BN), 0)
    ohf = (jnp.broadcast_to(lbl, (_C, _BN)) == cid).astype(jnp.float32)
    nsq = lax.dot_general(
        jnp.ones((1, _K), jnp.bfloat16), eb * eb, (((1,), (0,)), ((), ())),
        preferred_element_type=jnp.float32)
    r = 1.0 / jnp.maximum(jnp.sqrt(nsq), _EPS)
    g = 1.0 / (1.0 + jnp.exp(-jnp.sqrt(d2)))
    u = g * r
    B = jnp.concatenate([ohf, ohf * u], axis=0)
    sw_scr[...] += lax.dot_general(
        B.astype(jnp.bfloat16), eb, (((1,), (1,)), ((), ())),
        preferred_element_type=jnp.float32)
    cg_scr[0:_C, :] += jnp.broadcast_to(
        jnp.sum(ohf, axis=1, keepdims=True), (_C, _K))
    cg_scr[_C:2 * _C, :] += jnp.broadcast_to(
        jnp.sum(ohf * g, axis=1, keepdims=True), (_C, _K))

    @pl.when(j == _NB - 1)
    def _final():
        SW = sw_scr[...]
        S = SW[0:_C, :]
        W = SW[_C:2 * _C, :]
        cg = cg_scr[:, 0:1]
        cnt = cg[0:_C]
        gsum = cg[_C:2 * _C]
        present = cnt > 0.0
        invcnt = jnp.where(present, 1.0 / jnp.maximum(cnt, 1.0), 0.0)
        nsqS = jnp.sum(S * S, axis=1, keepdims=True)
        nm = jnp.sqrt(nsqS) * invcnt
        ci = lax.broadcasted_iota(jnp.int32, (_C, 1), 0)
        gate = jnp.logical_and(ci >= 1, present)
        a8 = jnp.where(gate, invcnt, 0.0)
        b8 = jnp.where(gate, invcnt * invcnt / jnp.maximum(nm, _EPS), 0.0)
        rowdot = jnp.sum(S * W, axis=1, keepdims=True)
        intra = jnp.sum(a8 * gsum) - jnp.sum(b8 * rowdot)
        Cm = lax.dot_general(
            S, S, (((1,), (1,)), ((), ())), preferred_element_type=jnp.float32)
        uu = cnt * jnp.maximum(nm, _EPS)
        v = jnp.where(gate, 1.0 / jnp.maximum(uu, _EPS * _EPS), 0.0)
        ri = lax.broadcasted_iota(jnp.int32, (_C, _C), 0)
        rj = lax.broadcasted_iota(jnp.int32, (_C, _C), 1)
        D = jnp.where(ri == rj, jnp.broadcast_to(v, (_C, _C)), 0.0)
        T1 = lax.dot_general(
            D, Cm, (((1,), (0,)), ((), ())), preferred_element_type=jnp.float32)
        T2 = lax.dot_general(
            T1, D, (((1,), (0,)), ((), ())), preferred_element_type=jnp.float32)
        inter = jnp.sum(jnp.where(ri != rj, T2, 0.0))
        Mn = jnp.sum(present.astype(jnp.float32))
        loss = intra / Mn + inter / (Mn * (Mn - 1.0))
        out_ref[...] = jnp.broadcast_to(loss, (1, 1))


def _tc_loss(E, lbl2, d22):
    return pl.pallas_call(
        _tc_body,
        grid=(_NB,),
        in_specs=[
            pl.BlockSpec((_K, _N), lambda j: (0, 0)),
            pl.BlockSpec((1, _N), lambda j: (0, 0)),
            pl.BlockSpec((1, _N), lambda j: (0, 0)),
        ],
        out_specs=pl.BlockSpec((1, 1), lambda j: (0, 0)),
        out_shape=jax.ShapeDtypeStruct((1, 1), jnp.float32),
        scratch_shapes=[
            pltpu.VMEM((2 * _C, _K), jnp.float32),
            pltpu.VMEM((2 * _C, _K), jnp.float32),
        ],
        compiler_params=pltpu.CompilerParams(
            dimension_semantics=("arbitrary",)),
    )(E, lbl2, d22)


def kernel(points, embedding, true):
    E = embedding.reshape(_K, _N).astype(jnp.bfloat16)
    lbl = true.reshape(_N).astype(jnp.int32)
    p0 = points.reshape(_N, 3).astype(jnp.float32)
    xs, ys, zs = p0[:, 0], p0[:, 1], p0[:, 2]
    d2 = _get_sc_d2()(xs, ys, zs, lbl)
    out = _tc_loss(E, lbl.reshape(1, _N), d2.reshape(1, _N))
    return out[0, 0]


# speedup vs baseline: 1.0929x; 1.0929x over previous
import functools

import jax
import jax.numpy as jnp
from jax import lax
from jax.experimental import pallas as pl
from jax.experimental.pallas import tpu as pltpu
from jax.experimental.pallas import tpu_sc as plsc

_N = 32768
_K = 128
_C = 8
_BN = 16384
_NB = _N // _BN
_EPS = 1e-8
_NS = 16
_CA = _N // _NS
_CB = _N // 32


def _sc_d2_body(xs_hbm, ys_hbm, zs_hbm, lbl_hbm, d2_hbm,
                xs_v, ys_v, zs_v, lbl_v, d2_v, tbl_v, mtab_v, pw_v, red_v,
                shared, sem):
    s = lax.axis_index("s")
    c = lax.axis_index("c")
    base_a = s * _CA
    sl_a = pl.ds(base_a, _CA)
    c1 = pltpu.async_copy(lbl_hbm.at[sl_a], lbl_v, sem)
    c2 = pltpu.async_copy(xs_hbm.at[sl_a], xs_v, sem)
    c3 = pltpu.async_copy(ys_hbm.at[sl_a], ys_v, sem)
    c4 = pltpu.async_copy(zs_hbm.at[sl_a], zs_v, sem)
    iota = lax.iota(jnp.int32, 16)
    zeros = jnp.zeros((16,), jnp.float32)
    ones = jnp.ones((16,), jnp.float32)

    def zbody(i, _):
        tbl_v[pl.ds(i * 16, 16)] = zeros
        return 0

    lax.fori_loop(0, 32, zbody, 0)
    c1.wait()
    c2.wait()
    c3.wait()
    c4.wait()

    def abody(i, _):
        sl = pl.ds(i * 16, 16)
        lbl = lbl_v[sl]
        x = xs_v[sl]
        y = ys_v[sl]
        z = zs_v[sl]
        idx = lbl * 16 + iota
        plsc.addupdate_scatter(tbl_v, [idx], ones)
        plsc.addupdate_scatter(tbl_v, [idx + 128], x)
        plsc.addupdate_scatter(tbl_v, [idx + 256], y)
        plsc.addupdate_scatter(tbl_v, [idx + 384], z)
        return 0

    lax.fori_loop(0, _CA // 16, abody, 0)

    base16 = (iota & 7) * 16 + jnp.where(iota >= 8, 128, 0)
    pw0 = zeros
    pw1 = zeros
    for jj in range(16):
        pw0 = pw0 + plsc.load_gather(tbl_v, [base16 + jj])
        pw1 = pw1 + plsc.load_gather(tbl_v, [base16 + (jj + 256)])
    pw_v[pl.ds(0, 16)] = pw0
    pw_v[pl.ds(16, 16)] = pw1
    pltpu.sync_copy(pw_v, shared.at[pl.ds(s * 32, 32)])
    plsc.subcore_barrier()
    pltpu.sync_copy(shared, red_v)

    def rbody(w, carry):
        a0, a1 = carry
        a0 = a0 + red_v[pl.ds(w * 32, 16)]
        a1 = a1 + red_v[pl.ds(w * 32 + 16, 16)]
        return (a0, a1)

    acc0, acc1 = lax.fori_loop(0, _NS, rbody, (zeros, zeros))
    mtab_v[pl.ds(0, 16)] = acc0
    mtab_v[pl.ds(16, 16)] = acc1
    lo = iota & 7
    cnt_rep = plsc.load_gather(mtab_v, [lo])
    sx_rep = plsc.load_gather(mtab_v, [lo + 8])
    sy_rep = plsc.load_gather(mtab_v, [lo + 16])
    sz_rep = plsc.load_gather(mtab_v, [lo + 24])
    pos = cnt_rep > 0.0
    safe = jnp.where(pos, cnt_rep, ones)
    mtab_v[pl.ds(32, 16)] = jnp.where(pos, sx_rep / safe, zeros)
    mtab_v[pl.ds(48, 16)] = jnp.where(pos, sy_rep / safe, zeros)
    mtab_v[pl.ds(64, 16)] = jnp.where(pos, sz_rep / safe, zeros)

    off = c * _CB

    def bbody(i, _):
        sl = pl.ds(off + i * 16, 16)
        lbl = lbl_v[sl]
        x = xs_v[sl]
        y = ys_v[sl]
        z = zs_v[sl]
        mx = plsc.load_gather(mtab_v, [lbl + 32])
        my = plsc.load_gather(mtab_v, [lbl + 48])
        mz = plsc.load_gather(mtab_v, [lbl + 64])
        dx = x - mx
        dy = y - my
        dz = z - mz
        d2_v[pl.ds(i * 16, 16)] = dx * dx + dy * dy + dz * dz
        return 0

    lax.fori_loop(0, _CB // 16, bbody, 0)
    pltpu.sync_copy(d2_v, d2_hbm.at[pl.ds(base_a + off, _CB)])


@functools.lru_cache(maxsize=1)
def _get_sc_d2():
    return pl.kernel(
        _sc_d2_body,
        out_type=jax.ShapeDtypeStruct((_N,), jnp.float32),
        mesh=plsc.VectorSubcoreMesh(core_axis_name="c", subcore_axis_name="s"),
        scratch_types=[
            pltpu.VMEM((_CA,), jnp.float32),
            pltpu.VMEM((_CA,), jnp.float32),
            pltpu.VMEM((_CA,), jnp.float32),
            pltpu.VMEM((_CA,), jnp.int32),
            pltpu.VMEM((_CB,), jnp.float32),
            pltpu.VMEM((512,), jnp.float32),
            pltpu.VMEM((96,), jnp.float32),
            pltpu.VMEM((32,), jnp.float32),
            pltpu.VMEM((512,), jnp.float32),
            pltpu.VMEM_SHARED((512,), jnp.float32),
            pltpu.SemaphoreType.DMA,
        ],
        compiler_params=pltpu.CompilerParams(needs_layout_passes=False),
    )


def _tc_body(e_ref, lbl_ref, d2_ref, out_ref, sw_scr, cg_scr):
    j = pl.program_id(0)

    @pl.when(j == 0)
    def _init():
        sw_scr[...] = jnp.zeros_like(sw_scr)
        cg_scr[...] = jnp.zeros_like(cg_scr)

    jstart = pl.multiple_of(j * _BN, _BN)
    e = e_ref[:, pl.ds(jstart, _BN)]
    lbl = lbl_ref[:, pl.ds(jstart, _BN)]
    d2 = d2_ref[:, pl.ds(jstart, _BN)]
    cid = lax.broadcasted_iota(jnp.int32, (_C, _BN), 0)
    ohf = (jnp.broadcast_to(lbl, (_C, _BN)) == cid).astype(jnp.float32)
    eb = e.astype(jnp.bfloat16)
    nsq = lax.dot_general(
        jnp.ones((1, _K), jnp.bfloat16), eb * eb, (((1,), (0,)), ((), ())),
        preferred_element_type=jnp.float32)
    r = 1.0 / jnp.maximum(jnp.sqrt(nsq), _EPS)
    g = 1.0 / (1.0 + jnp.exp(-jnp.sqrt(d2)))
    u = g * r
    B = jnp.concatenate([ohf, ohf * u], axis=0)
    sw_scr[...] += lax.dot_general(
        B.astype(jnp.bfloat16), eb, (((1,), (1,)), ((), ())),
        preferred_element_type=jnp.float32)
    cg_scr[0:_C, :] += jnp.broadcast_to(
        jnp.sum(ohf, axis=1, keepdims=True), (_C, _K))
    cg_scr[_C:2 * _C, :] += jnp.broadcast_to(
        jnp.sum(ohf * g, axis=1, keepdims=True), (_C, _K))

    @pl.when(j == _NB - 1)
    def _final():
        SW = sw_scr[...]
        S = SW[0:_C, :]
        W = SW[_C:2 * _C, :]
        cg = cg_scr[:, 0:1]
        cnt = cg[0:_C]
        gsum = cg[_C:2 * _C]
        present = cnt > 0.0
        invcnt = jnp.where(present, 1.0 / jnp.maximum(cnt, 1.0), 0.0)
        nsqS = jnp.sum(S * S, axis=1, keepdims=True)
        nm = jnp.sqrt(nsqS) * invcnt
        ci = lax.broadcasted_iota(jnp.int32, (_C, 1), 0)
        gate = jnp.logical_and(ci >= 1, present)
        a8 = jnp.where(gate, invcnt, 0.0)
        b8 = jnp.where(gate, invcnt * invcnt / jnp.maximum(nm, _EPS), 0.0)
        rowdot = jnp.sum(S * W, axis=1, keepdims=True)
        intra = jnp.sum(a8 * gsum) - jnp.sum(b8 * rowdot)
        Cm = lax.dot_general(
            S, S, (((1,), (1,)), ((), ())), preferred_element_type=jnp.float32)
        uu = cnt * jnp.maximum(nm, _EPS)
        v = jnp.where(gate, 1.0 / jnp.maximum(uu, _EPS * _EPS), 0.0)
        ri = lax.broadcasted_iota(jnp.int32, (_C, _C), 0)
        rj = lax.broadcasted_iota(jnp.int32, (_C, _C), 1)
        D = jnp.where(ri == rj, jnp.broadcast_to(v, (_C, _C)), 0.0)
        T1 = lax.dot_general(
            D, Cm, (((1,), (0,)), ((), ())), preferred_element_type=jnp.float32)
        T2 = lax.dot_general(
            T1, D, (((1,), (0,)), ((), ())), preferred_element_type=jnp.float32)
        inter = jnp.sum(jnp.where(ri != rj, T2, 0.0))
        Mn = jnp.sum(present.astype(jnp.float32))
        loss = intra / Mn + inter / (Mn * (Mn - 1.0))
        out_ref[...] = jnp.broadcast_to(loss, (1, 1))


def _tc_loss(E, lbl2, d22):
    return pl.pallas_call(
        _tc_body,
        grid=(_NB,),
        in_specs=[
            pl.BlockSpec((_K, _N), lambda j: (0, 0)),
            pl.BlockSpec((1, _N), lambda j: (0, 0)),
            pl.BlockSpec((1, _N), lambda j: (0, 0)),
        ],
        out_specs=pl.BlockSpec((1, 1), lambda j: (0, 0)),
        out_shape=jax.ShapeDtypeStruct((1, 1), jnp.float32),
        scratch_shapes=[
            pltpu.VMEM((2 * _C, _K), jnp.float32),
            pltpu.VMEM((2 * _C, _K), jnp.float32),
        ],
        compiler_params=pltpu.CompilerParams(
            dimension_semantics=("arbitrary",)),
    )(E, lbl2, d22)


def kernel(points, embedding, true):
    E = embedding.reshape(_K, _N)
    lbl = true.reshape(_N).astype(jnp.int32)
    p0 = points.reshape(_N, 3).astype(jnp.float32)
    xs, ys, zs = p0[:, 0], p0[:, 1], p0[:, 2]
    d2 = _get_sc_d2()(xs, ys, zs, lbl)
    out = _tc_loss(E, lbl.reshape(1, _N), d2.reshape(1, _N))
    return out[0, 0]
